# baseline (speedup 1.0000x reference)
import functools

import jax
import jax.numpy as jnp
from jax import lax
from jax.experimental import pallas as pl
from jax.experimental.pallas import tpu as pltpu
from jax.experimental.pallas import tpu_sc as plsc

N = 10000
E = 320000
F = 128
H = 128
C = 40

NC = 2
NS = 16
L = 16
NW = NC * NS
EPW = E // NW
CH = 100
NCHUNK = EPW // CH
RPT = N // NS
FP = F + L


def _sc_aggregate(xp, ei):
    mesh = plsc.VectorSubcoreMesh(core_axis_name="c", subcore_axis_name="s")

    @functools.partial(
        pl.kernel,
        out_type=jax.ShapeDtypeStruct((NC, N, FP), jnp.float32),
        mesh=mesh,
        compiler_params=pltpu.CompilerParams(use_tc_tiling_on_sc=False),
        scratch_types=[
            pltpu.VMEM((4, 2, CH), jnp.int32),
            pltpu.VMEM((CH, FP), jnp.float32),
            pltpu.VMEM((CH, FP), jnp.float32),
            pltpu.VMEM_SHARED((N, FP), jnp.float32),
            pltpu.SemaphoreType.DMA,
            pltpu.SemaphoreType.DMA,
            pltpu.SemaphoreType.DMA,
            pltpu.SemaphoreType.DMA,
            pltpu.SemaphoreType.DMA,
            pltpu.SemaphoreType.DMA,
            pltpu.SemaphoreType.DMA,
            pltpu.SemaphoreType.DMA,
        ],
    )
    def agg(xp_hbm, ei_hbm, acc_out,
            idx, rowsA, rowsB, acc_sh,
            sGA, sGB, sSA, sSB, sI0, sI1, sI2, sI3):
        sI = (sI0, sI1, sI2, sI3)
        cid = lax.axis_index("c")
        sid = lax.axis_index("s")
        wid = sid * NC + cid

        zv = jnp.zeros((L,), jnp.float32)

        @pl.loop(0, CH)
        def _(i):
            for j in range(FP // L):
                rowsA[i, pl.ds(j * L, L)] = zv

        r0 = sid * RPT
        for t in range(RPT // CH):
            pltpu.sync_copy(rowsA, acc_sh.at[pl.ds(r0 + t * CH, CH)])
        pltpu.sync_copy(rowsA.at[pl.ds(0, RPT % CH)],
                        acc_sh.at[pl.ds(r0 + (RPT // CH) * CH, RPT % CH)])

        for j in range(4):
            pltpu.async_copy(ei_hbm.at[wid, j], idx.at[j], sI[j])

        def g_wait(sem, buf):
            pltpu.make_async_copy(xp_hbm.at[idx.at[0, 0]], buf, sem).wait()

        def s_wait(sem, buf):
            pltpu.make_async_copy(buf, acc_sh.at[idx.at[0, 1]], sem).wait()

        def i_wait(j):
            pltpu.make_async_copy(ei_hbm.at[wid, 0], idx.at[j], sI[j]).wait()

        i_wait(0)
        pltpu.async_copy(xp_hbm.at[idx.at[0, 0]], rowsA, sGA)
        i_wait(1)
        pltpu.async_copy(xp_hbm.at[idx.at[1, 0]], rowsB, sGB)

        plsc.subcore_barrier()

        @pl.loop(0, NCHUNK, step=4)
        def _(k):
            g_wait(sGA, rowsA)
            pltpu.async_copy(rowsA, acc_sh.at[idx.at[0, 1]], sSA, add=True)
            g_wait(sGB, rowsB)
            pltpu.async_copy(rowsB, acc_sh.at[idx.at[1, 1]], sSB, add=True)

            s_wait(sSA, rowsA)

            @pl.when(k + 4 < NCHUNK)
            def _():
                pltpu.async_copy(ei_hbm.at[wid, k + 4], idx.at[0], sI0)

            i_wait(2)
            pltpu.async_copy(xp_hbm.at[idx.at[2, 0]], rowsA, sGA)

            s_wait(sSB, rowsB)

            @pl.when(k + 5 < NCHUNK)
            def _():
                pltpu.async_copy(ei_hbm.at[wid, k + 5], idx.at[1], sI1)

            i_wait(3)
            pltpu.async_copy(xp_hbm.at[idx.at[3, 0]], rowsB, sGB)

            g_wait(sGA, rowsA)
            pltpu.async_copy(rowsA, acc_sh.at[idx.at[2, 1]], sSA, add=True)
            g_wait(sGB, rowsB)
            pltpu.async_copy(rowsB, acc_sh.at[idx.at[3, 1]], sSB, add=True)

            s_wait(sSA, rowsA)

            @pl.when(k + 6 < NCHUNK)
            def _():
                pltpu.async_copy(ei_hbm.at[wid, k + 6], idx.at[2], sI2)

            @pl.when(k + 4 < NCHUNK)
            def _():
                i_wait(0)
                pltpu.async_copy(xp_hbm.at[idx.at[0, 0]], rowsA, sGA)

            s_wait(sSB, rowsB)

            @pl.when(k + 7 < NCHUNK)
            def _():
                pltpu.async_copy(ei_hbm.at[wid, k + 7], idx.at[3], sI3)

            @pl.when(k + 5 < NCHUNK)
            def _():
                i_wait(1)
                pltpu.async_copy(xp_hbm.at[idx.at[1, 0]], rowsB, sGB)

        plsc.subcore_barrier()

        pltpu.sync_copy(acc_sh.at[pl.ds(r0, RPT)],
                        acc_out.at[cid, pl.ds(r0, RPT)])

    return agg(xp, ei)


def _tc_dense(x, acc, W_self, W_neigh, b2, W_fc, bf2):
    R = 1000

    def body(x_ref, p_ref, ws_ref, wn_ref, b_ref, wf_ref, bf_ref, o_ref):
        dot = functools.partial(jnp.dot,
                                preferred_element_type=jnp.float32,
                                precision=lax.Precision.HIGHEST)
        s = p_ref[0, :, :F] + p_ref[1, :, :F]
        deg = p_ref[0, :, F] + p_ref[1, :, F]
        neigh = s / jnp.maximum(deg, 1.0)[:, None]
        h = dot(x_ref[...], ws_ref[...]) + dot(neigh, wn_ref[...]) + b_ref[...]
        h = jnp.maximum(h, 0.0)
        o_ref[...] = dot(h, wf_ref[...]) + bf_ref[...]

    return pl.pallas_call(
        body,
        grid=(N // R,),
        in_specs=[
            pl.BlockSpec((R, F), lambda i: (i, 0)),
            pl.BlockSpec((NC, R, FP), lambda i: (0, i, 0)),
            pl.BlockSpec((F, H), lambda i: (0, 0)),
            pl.BlockSpec((F, H), lambda i: (0, 0)),
            pl.BlockSpec((1, H), lambda i: (0, 0)),
            pl.BlockSpec((H, C), lambda i: (0, 0)),
            pl.BlockSpec((1, C), lambda i: (0, 0)),
        ],
        out_specs=pl.BlockSpec((R, C), lambda i: (i, 0)),
        out_shape=jax.ShapeDtypeStruct((N, C), jnp.float32),
    )(x, acc, W_self, W_neigh, b2, W_fc, bf2)


def kernel(x, edge_index, W_self, W_neigh, b, W_fc, b_fc):
    xp = jnp.concatenate([x, jnp.ones((N, L), jnp.float32)], axis=1)
    src = edge_index[0].astype(jnp.int32).reshape(NW, NCHUNK, CH)
    dst = edge_index[1].astype(jnp.int32).reshape(NW, NCHUNK, CH)
    ei = jnp.stack([src, dst], axis=2)
    acc = _sc_aggregate(xp, ei)
    return _tc_dense(x, acc, W_self, W_neigh,
                     b.reshape(1, H), W_fc, b_fc.reshape(1, C))


# speedup vs baseline: 1.1468x; 1.1468x over previous
import functools

import jax
import jax.numpy as jnp
from jax import lax
from jax.experimental import pallas as pl
from jax.experimental.pallas import tpu as pltpu
from jax.experimental.pallas import tpu_sc as plsc

N = 10000
E = 320000
F = 128
H = 128
C = 40

NC = 2
NS = 16
L = 16
NW = NC * NS
EPW = E // NW
CH = 100
NCHUNK = EPW // CH
RPT = N // NS
FP = F + L


def _sc_aggregate(xp, src4, dst4):
    mesh = plsc.VectorSubcoreMesh(core_axis_name="c", subcore_axis_name="s")

    @functools.partial(
        pl.kernel,
        out_type=jax.ShapeDtypeStruct((NC, N, FP), jnp.float32),
        mesh=mesh,
        compiler_params=pltpu.CompilerParams(use_tc_tiling_on_sc=False),
        scratch_types=[
            pltpu.VMEM((4, 2, CH), jnp.int32),
            pltpu.VMEM((CH, FP), jnp.float32),
            pltpu.VMEM((CH, FP), jnp.float32),
            pltpu.VMEM_SHARED((N, FP), jnp.float32),
            pltpu.SemaphoreType.DMA,
            pltpu.SemaphoreType.DMA,
            pltpu.SemaphoreType.DMA,
            pltpu.SemaphoreType.DMA,
            pltpu.SemaphoreType.DMA,
            pltpu.SemaphoreType.DMA,
            pltpu.SemaphoreType.DMA,
            pltpu.SemaphoreType.DMA,
        ],
    )
    def agg(src_hbm, dst_hbm, xp_hbm, acc_out,
            idx, rowsA, rowsB, acc_sh,
            sGA, sGB, sSA, sSB, sI0, sI1, sI2, sI3):
        sI = (sI0, sI1, sI2, sI3)
        cid = lax.axis_index("c")
        sid = lax.axis_index("s")
        wid = sid * NC + cid

        zv = jnp.zeros((L,), jnp.float32)

        @pl.loop(0, CH)
        def _(i):
            for j in range(FP // L):
                rowsA[i, pl.ds(j * L, L)] = zv

        r0 = sid * RPT
        for t in range(RPT // CH):
            pltpu.sync_copy(rowsA, acc_sh.at[pl.ds(r0 + t * CH, CH)])
        pltpu.sync_copy(rowsA.at[pl.ds(0, RPT % CH)],
                        acc_sh.at[pl.ds(r0 + (RPT // CH) * CH, RPT % CH)])

        def i_fetch(c, j):
            pltpu.async_copy(src_hbm.at[wid, c], idx.at[j, 0], sI[j])
            pltpu.async_copy(dst_hbm.at[wid, c], idx.at[j, 1], sI[j])

        def g_wait(sem, buf):
            pltpu.make_async_copy(xp_hbm.at[idx.at[0, 0]], buf, sem).wait()

        def s_wait(sem, buf):
            pltpu.make_async_copy(buf, acc_sh.at[idx.at[0, 1]], sem).wait()

        def i_wait(j):
            pltpu.make_async_copy(src_hbm.at[wid, 0], idx.at[j, 0],
                                  sI[j]).wait()
            pltpu.make_async_copy(src_hbm.at[wid, 0], idx.at[j, 1],
                                  sI[j]).wait()

        for j in range(4):
            i_fetch(j, j)

        i_wait(0)
        pltpu.async_copy(xp_hbm.at[idx.at[0, 0]], rowsA, sGA)
        i_wait(1)
        pltpu.async_copy(xp_hbm.at[idx.at[1, 0]], rowsB, sGB)

        plsc.subcore_barrier()

        @pl.loop(0, NCHUNK, step=4)
        def _(k):
            g_wait(sGA, rowsA)
            pltpu.async_copy(rowsA, acc_sh.at[idx.at[0, 1]], sSA, add=True)
            g_wait(sGB, rowsB)
            pltpu.async_copy(rowsB, acc_sh.at[idx.at[1, 1]], sSB, add=True)

            s_wait(sSA, rowsA)

            @pl.when(k + 4 < NCHUNK)
            def _():
                i_fetch(k + 4, 0)

            i_wait(2)
            pltpu.async_copy(xp_hbm.at[idx.at[2, 0]], rowsA, sGA)

            s_wait(sSB, rowsB)

            @pl.when(k + 5 < NCHUNK)
            def _():
                i_fetch(k + 5, 1)

            i_wait(3)
            pltpu.async_copy(xp_hbm.at[idx.at[3, 0]], rowsB, sGB)

            g_wait(sGA, rowsA)
            pltpu.async_copy(rowsA, acc_sh.at[idx.at[2, 1]], sSA, add=True)
            g_wait(sGB, rowsB)
            pltpu.async_copy(rowsB, acc_sh.at[idx.at[3, 1]], sSB, add=True)

            s_wait(sSA, rowsA)

            @pl.when(k + 6 < NCHUNK)
            def _():
                i_fetch(k + 6, 2)

            @pl.when(k + 4 < NCHUNK)
            def _():
                i_wait(0)
                pltpu.async_copy(xp_hbm.at[idx.at[0, 0]], rowsA, sGA)

            s_wait(sSB, rowsB)

            @pl.when(k + 7 < NCHUNK)
            def _():
                i_fetch(k + 7, 3)

            @pl.when(k + 5 < NCHUNK)
            def _():
                i_wait(1)
                pltpu.async_copy(xp_hbm.at[idx.at[1, 0]], rowsB, sGB)

        plsc.subcore_barrier()

        pltpu.sync_copy(acc_sh.at[pl.ds(r0, RPT)],
                        acc_out.at[cid, pl.ds(r0, RPT)])

    return agg(src4, dst4, xp)


def _tc_dense(x, acc, W_self, W_neigh, b2, W_fc, bf2):
    R = 1000

    def body(x_ref, p_ref, ws_ref, wn_ref, b_ref, wf_ref, bf_ref, o_ref):
        dot = functools.partial(jnp.dot,
                                preferred_element_type=jnp.float32,
                                precision=lax.Precision.DEFAULT)
        s = p_ref[0, :, :F] + p_ref[1, :, :F]
        deg = p_ref[0, :, F] + p_ref[1, :, F]
        neigh = s / jnp.maximum(deg, 1.0)[:, None]
        h = dot(x_ref[...], ws_ref[...]) + dot(neigh, wn_ref[...]) + b_ref[...]
        h = jnp.maximum(h, 0.0)
        o_ref[...] = dot(h, wf_ref[...]) + bf_ref[...]

    return pl.pallas_call(
        body,
        grid=(N // R,),
        in_specs=[
            pl.BlockSpec((R, F), lambda i: (i, 0)),
            pl.BlockSpec((NC, R, FP), lambda i: (0, i, 0)),
            pl.BlockSpec((F, H), lambda i: (0, 0)),
            pl.BlockSpec((F, H), lambda i: (0, 0)),
            pl.BlockSpec((1, H), lambda i: (0, 0)),
            pl.BlockSpec((H, C), lambda i: (0, 0)),
            pl.BlockSpec((1, C), lambda i: (0, 0)),
        ],
        out_specs=pl.BlockSpec((R, C), lambda i: (i, 0)),
        out_shape=jax.ShapeDtypeStruct((N, C), jnp.float32),
    )(x, acc, W_self, W_neigh, b2, W_fc, bf2)


def kernel(x, edge_index, W_self, W_neigh, b, W_fc, b_fc):
    xp = jnp.concatenate([x, jnp.ones((N, L), jnp.float32)], axis=1)
    src4 = edge_index[0].astype(jnp.int32).reshape(NW, NCHUNK, CH)
    dst4 = edge_index[1].astype(jnp.int32).reshape(NW, NCHUNK, CH)
    acc = _sc_aggregate(xp, src4, dst4)
    return _tc_dense(x, acc, W_self, W_neigh,
                     b.reshape(1, H), W_fc, b_fc.reshape(1, C))


# speedup vs baseline: 1.2638x; 1.1021x over previous
import functools

import jax
import jax.numpy as jnp
from jax import lax
from jax.experimental import pallas as pl
from jax.experimental.pallas import tpu as pltpu
from jax.experimental.pallas import tpu_sc as plsc

N = 10000
E = 320000
F = 128
H = 128
C = 40

NC = 2
NS = 16
L = 16
NW = NC * NS
EPW = E // NW
CH = 80
NCHUNK = EPW // CH
RPT = N // NS
FP = F + L


def _sc_aggregate(xp, ei):
    mesh = plsc.VectorSubcoreMesh(core_axis_name="c", subcore_axis_name="s")

    @functools.partial(
        pl.kernel,
        out_type=jax.ShapeDtypeStruct((NC, N, FP), jnp.float32),
        mesh=mesh,
        compiler_params=pltpu.CompilerParams(use_tc_tiling_on_sc=False),
        scratch_types=[
            pltpu.VMEM((4, 2, CH), jnp.int32),
            pltpu.VMEM((CH, FP), jnp.float32),
            pltpu.VMEM((CH, FP), jnp.float32),
            pltpu.VMEM_SHARED((N, FP), jnp.float32),
            pltpu.SemaphoreType.DMA,
            pltpu.SemaphoreType.DMA,
            pltpu.SemaphoreType.DMA,
            pltpu.SemaphoreType.DMA,
            pltpu.SemaphoreType.DMA,
            pltpu.SemaphoreType.DMA,
            pltpu.SemaphoreType.DMA,
            pltpu.SemaphoreType.DMA,
        ],
    )
    def agg(ei_hbm, xp_hbm, acc_out,
            idx, rowsA, rowsB, acc_sh,
            sGA, sGB, sSA, sSB, sI0, sI1, sI2, sI3):
        sI = (sI0, sI1, sI2, sI3)
        cid = lax.axis_index("c")
        sid = lax.axis_index("s")
        wid = sid * NC + cid

        zv = jnp.zeros((L,), jnp.float32)

        @pl.loop(0, CH)
        def _(i):
            for j in range(FP // L):
                rowsA[i, pl.ds(j * L, L)] = zv

        r0 = sid * RPT
        for t in range(RPT // CH):
            pltpu.sync_copy(rowsA, acc_sh.at[pl.ds(r0 + t * CH, CH)])
        pltpu.sync_copy(rowsA.at[pl.ds(0, RPT % CH)],
                        acc_sh.at[pl.ds(r0 + (RPT // CH) * CH, RPT % CH)])

        def i_fetch(c, j):
            off = wid * EPW + c * CH
            pltpu.async_copy(ei_hbm.at[0, pl.ds(off, CH)], idx.at[j, 0], sI[j])
            pltpu.async_copy(ei_hbm.at[1, pl.ds(off, CH)], idx.at[j, 1], sI[j])

        def g_wait(sem, buf):
            pltpu.make_async_copy(xp_hbm.at[idx.at[0, 0]], buf, sem).wait()

        def s_wait(sem, buf):
            pltpu.make_async_copy(buf, acc_sh.at[idx.at[0, 1]], sem).wait()

        def i_wait(j):
            pltpu.make_async_copy(ei_hbm.at[0, pl.ds(0, CH)], idx.at[j, 0],
                                  sI[j]).wait()
            pltpu.make_async_copy(ei_hbm.at[0, pl.ds(0, CH)], idx.at[j, 1],
                                  sI[j]).wait()

        for j in range(4):
            i_fetch(j, j)

        i_wait(0)
        pltpu.async_copy(xp_hbm.at[idx.at[0, 0]], rowsA, sGA)
        i_wait(1)
        pltpu.async_copy(xp_hbm.at[idx.at[1, 0]], rowsB, sGB)

        plsc.subcore_barrier()

        @pl.loop(0, NCHUNK - 1, step=4)
        def _(k):
            g_wait(sGA, rowsA)
            pltpu.async_copy(rowsA, acc_sh.at[idx.at[0, 1]], sSA, add=True)
            g_wait(sGB, rowsB)
            pltpu.async_copy(rowsB, acc_sh.at[idx.at[1, 1]], sSB, add=True)

            s_wait(sSA, rowsA)

            @pl.when(k + 4 < NCHUNK)
            def _():
                i_fetch(k + 4, 0)

            i_wait(2)
            pltpu.async_copy(xp_hbm.at[idx.at[2, 0]], rowsA, sGA)

            s_wait(sSB, rowsB)

            @pl.when(k + 5 < NCHUNK)
            def _():
                i_fetch(k + 5, 1)

            i_wait(3)
            pltpu.async_copy(xp_hbm.at[idx.at[3, 0]], rowsB, sGB)

            g_wait(sGA, rowsA)
            pltpu.async_copy(rowsA, acc_sh.at[idx.at[2, 1]], sSA, add=True)
            g_wait(sGB, rowsB)
            pltpu.async_copy(rowsB, acc_sh.at[idx.at[3, 1]], sSB, add=True)

            s_wait(sSA, rowsA)

            @pl.when(k + 6 < NCHUNK)
            def _():
                i_fetch(k + 6, 2)

            @pl.when(k + 4 < NCHUNK)
            def _():
                i_wait(0)
                pltpu.async_copy(xp_hbm.at[idx.at[0, 0]], rowsA, sGA)

            s_wait(sSB, rowsB)

            @pl.when(k + 7 < NCHUNK)
            def _():
                i_fetch(k + 7, 3)

            @pl.when(k + 5 < NCHUNK)
            def _():
                i_wait(1)
                pltpu.async_copy(xp_hbm.at[idx.at[1, 0]], rowsB, sGB)

        g_wait(sGA, rowsA)
        pltpu.async_copy(rowsA, acc_sh.at[idx.at[0, 1]], sSA, add=True)
        s_wait(sSA, rowsA)

        plsc.subcore_barrier()

        pltpu.sync_copy(acc_sh.at[pl.ds(r0, RPT)],
                        acc_out.at[cid, pl.ds(r0, RPT)])

    return agg(ei, xp)


def _tc_dense(x, acc, W_self, W_neigh, b2, W_fc, bf2):
    R = 1000

    def body(x_ref, p_ref, ws_ref, wn_ref, b_ref, wf_ref, bf_ref, o_ref):
        dot = functools.partial(jnp.dot,
                                preferred_element_type=jnp.float32,
                                precision=lax.Precision.DEFAULT)
        s = p_ref[0, :, :F] + p_ref[1, :, :F]
        deg = p_ref[0, :, F] + p_ref[1, :, F]
        neigh = s / jnp.maximum(deg, 1.0)[:, None]
        h = dot(x_ref[...], ws_ref[...]) + dot(neigh, wn_ref[...]) + b_ref[...]
        h = jnp.maximum(h, 0.0)
        o_ref[...] = dot(h, wf_ref[...]) + bf_ref[...]

    return pl.pallas_call(
        body,
        grid=(N // R,),
        in_specs=[
            pl.BlockSpec((R, F), lambda i: (i, 0)),
            pl.BlockSpec((NC, R, FP), lambda i: (0, i, 0)),
            pl.BlockSpec((F, H), lambda i: (0, 0)),
            pl.BlockSpec((F, H), lambda i: (0, 0)),
            pl.BlockSpec((1, H), lambda i: (0, 0)),
            pl.BlockSpec((H, C), lambda i: (0, 0)),
            pl.BlockSpec((1, C), lambda i: (0, 0)),
        ],
        out_specs=pl.BlockSpec((R, C), lambda i: (i, 0)),
        out_shape=jax.ShapeDtypeStruct((N, C), jnp.float32),
    )(x, acc, W_self, W_neigh, b2, W_fc, bf2)


def kernel(x, edge_index, W_self, W_neigh, b, W_fc, b_fc):
    xp = jnp.concatenate([x, jnp.ones((N, L), jnp.float32)], axis=1)
    acc = _sc_aggregate(xp, edge_index.astype(jnp.int32))
    return _tc_dense(x, acc, W_self, W_neigh,
                     b.reshape(1, H), W_fc, b_fc.reshape(1, C))
